# baseline (speedup 1.0000x reference)
import dataclasses
import functools

import jax
import jax.numpy as jnp
from jax import lax
from jax.experimental import pallas as pl
from jax.experimental.pallas import tpu as pltpu
from jax.experimental.pallas import tpu_sc as plsc

N = 10000
D = 128
E = 320000
NC = 2
NS = 16
NW = NC * NS
EPW = E // NW
CHUNK = 80
NCHUNK = EPW // CHUNK
EPW2 = 10240
EPADDED = EPW2 * NW
CHUNK2 = 128
NCHUNK2 = EPW2 // CHUNK2
NPAD = 10240
RPS = NPAD // NS
PR = NPAD // 128
NBUF = 2

_MESH = plsc.VectorSubcoreMesh(core_axis_name="c", subcore_axis_name="s")


def _make_sc_sums(with_hist, CHUNK, NCHUNK):
    def body(x_hbm, src_hbm, dst_hbm, z_hbm, id_hbm, psum_hbm, pcnt_hbm,
             didx_all, sidx, rows, acc, hist, iid, cacc, isem, gsem, ssem):
        cid = lax.axis_index("c")
        sid = lax.axis_index("s")
        wid = sid * NC + cid
        r0 = sid * RPS
        pltpu.sync_copy(z_hbm.at[pl.ds(r0, RPS)], acc.at[pl.ds(r0, RPS)])
        pltpu.sync_copy(dst_hbm.at[wid], didx_all)
        if with_hist:
            pltpu.sync_copy(z_hbm.at[pl.ds(0, PR)], hist)
            pltpu.sync_copy(id_hbm.at[0], iid)

            @pl.when(sid == 0)
            def _():
                pltpu.sync_copy(z_hbm.at[pl.ds(0, PR)], cacc)
        plsc.subcore_barrier()

        ones16 = jnp.full((16,), 1.0, jnp.float32)

        def step(c, b):
            pltpu.make_async_copy(x_hbm.at[sidx[b].at[0]], rows[b],
                                  gsem[b]).wait()
            pltpu.async_copy(rows[b], acc.at[didx_all.at[c]], ssem[b],
                             add=True)

            @pl.when(c + NBUF < NCHUNK)
            def _():
                pltpu.async_copy(src_hbm.at[wid, c + NBUF], sidx[b], isem[b])

            if with_hist:
                @pl.loop(0, CHUNK, step=16)
                def _(k0):
                    v = didx_all[c, pl.ds(k0, 16)]
                    plsc.addupdate_scatter(
                        hist, [lax.shift_right_logical(v, 7),
                               lax.bitwise_and(v, 127)], ones16)

            b1 = (b + 1) % NBUF

            @pl.when(c + 1 < NCHUNK)
            def _():
                @pl.when(c + 1 >= NBUF)
                def _():
                    pltpu.make_async_copy(rows[b1],
                                          acc.at[didx_all.at[c + 1 - NBUF]],
                                          ssem[b1]).wait()
                pltpu.make_async_copy(src_hbm.at[wid, c + 1], sidx[b1],
                                      isem[b1]).wait()
                pltpu.async_copy(x_hbm.at[sidx[b1].at[0]], rows[b1], gsem[b1])

        for k in range(NBUF):
            pltpu.async_copy(src_hbm.at[wid, k], sidx[k], isem[k])
        pltpu.make_async_copy(src_hbm.at[wid, 0], sidx[0], isem[0]).wait()
        pltpu.async_copy(x_hbm.at[sidx[0].at[0]], rows[0], gsem[0])

        NMAIN = (NCHUNK // NBUF) * NBUF

        @pl.loop(0, NMAIN, step=NBUF)
        def _(t):
            for k in range(NBUF):
                step(t + k, k)

        for c in range(NMAIN, NCHUNK):
            @pl.loop(c, c + 1)
            def _(ct, _b=c % NBUF):
                step(ct, _b)

        for c in range(NCHUNK - NBUF, NCHUNK):
            @pl.loop(c, c + 1)
            def _(ct, _b=c % NBUF):
                pltpu.make_async_copy(rows[_b], acc.at[didx_all.at[ct]],
                                      ssem[_b]).wait()

        if with_hist:
            pltpu.sync_copy(hist, cacc.at[iid], add=True)

        plsc.subcore_barrier()
        pltpu.sync_copy(acc.at[pl.ds(r0, RPS)],
                        psum_hbm.at[cid, pl.ds(r0, RPS)])
        if with_hist:
            @pl.when(sid == 0)
            def _():
                pltpu.sync_copy(cacc, pcnt_hbm.at[cid])

    out_type = [jax.ShapeDtypeStruct((NC, NPAD, D), jnp.float32)]
    if with_hist:
        out_type.append(jax.ShapeDtypeStruct((NC, PR, 128), jnp.float32))
    hp = PR if with_hist else 8
    scratch = (
        [pltpu.VMEM((NCHUNK, CHUNK), jnp.int32)]
        + [pltpu.VMEM((1, CHUNK), jnp.int32)] * NBUF
        + [pltpu.VMEM((CHUNK, D), jnp.float32)] * NBUF
        + [pltpu.VMEM_SHARED((NPAD, D), jnp.float32)]
        + [pltpu.VMEM((hp, 128), jnp.float32),
           pltpu.VMEM((hp,), jnp.int32),
           pltpu.VMEM_SHARED((hp, 128), jnp.float32)]
        + [pltpu.SemaphoreType.DMA] * (3 * NBUF)
    )
    cp = pltpu.CompilerParams()
    if with_hist and (
            "needs_layout_passes" in pltpu.CompilerParams.__dataclass_fields__):
        cp = dataclasses.replace(cp, needs_layout_passes=False)

    def fn(x_hbm, src_hbm, dst_hbm, z_hbm, id_hbm, *rest):
        psum_hbm = rest[0]
        rest = rest[1:]
        if with_hist:
            pcnt_hbm = rest[0]
            rest = rest[1:]
        else:
            pcnt_hbm = None
        didx_all = rest[0]
        sidx = rest[1:1 + NBUF]
        rows = rest[1 + NBUF:1 + 2 * NBUF]
        acc, hist, iid, cacc = rest[1 + 2 * NBUF:5 + 2 * NBUF]
        sems = rest[5 + 2 * NBUF:]
        isem = sems[0:NBUF]
        gsem = sems[NBUF:2 * NBUF]
        ssem = sems[2 * NBUF:3 * NBUF]
        body(x_hbm, src_hbm, dst_hbm, z_hbm, id_hbm, psum_hbm, pcnt_hbm,
             didx_all, sidx, rows, acc, hist, iid, cacc, isem, gsem, ssem)

    return pl.kernel(fn, mesh=_MESH, out_type=out_type,
                     scratch_types=scratch, compiler_params=cp)


_sc_sums_hist = _make_sc_sums(True, CHUNK, NCHUNK)
_sc_sums = _make_sc_sums(False, CHUNK2, NCHUNK2)

BLK = 1024


def _tc_layer_body(relu, p_ref, pc_ref, x_ref, wl_ref, wr_ref, b_ref, o_ref):
    s = p_ref[0] + p_ref[1]
    cw = pc_ref[0] + pc_ref[1]
    rep = jnp.broadcast_to(cw[:, None, :], (BLK // 128, 128, 128))
    rep = rep.reshape(BLK, 128)
    row_mod = lax.broadcasted_iota(jnp.int32, (BLK, 128), 0) % 128
    lane = lax.broadcasted_iota(jnp.int32, (BLK, 128), 1)
    cnt = jnp.sum(jnp.where(row_mod == lane, rep, 0.0), axis=1, keepdims=True)
    mean = s / jnp.maximum(cnt, 1.0)
    acc = lax.dot_general(mean, wl_ref[...], (((1,), (1,)), ((), ())),
                          preferred_element_type=jnp.float32)
    acc += lax.dot_general(x_ref[...], wr_ref[...], (((1,), (1,)), ((), ())),
                           preferred_element_type=jnp.float32)
    acc += b_ref[...]
    o_ref[...] = jnp.maximum(acc, 0.0) if relu else acc


def _tc_layer(psum, pcnt, x, Wl, Wr, b, relu):
    grid = (NPAD // BLK,)
    return pl.pallas_call(
        functools.partial(_tc_layer_body, relu),
        grid=grid,
        in_specs=[
            pl.BlockSpec((NC, BLK, D), lambda i: (0, i, 0)),
            pl.BlockSpec((NC, BLK // 128, 128), lambda i: (0, i, 0)),
            pl.BlockSpec((BLK, D), lambda i: (i, 0)),
            pl.BlockSpec((D, D), lambda i: (0, 0)),
            pl.BlockSpec((D, D), lambda i: (0, 0)),
            pl.BlockSpec((1, D), lambda i: (0, 0)),
        ],
        out_specs=pl.BlockSpec((BLK, D), lambda i: (i, 0)),
        out_shape=jax.ShapeDtypeStruct((N, D), jnp.float32),
    )(psum, pcnt, x, Wl, Wr, b.reshape(1, D))


def kernel(x, edge_index, W1l, W1r, b1, W2l, W2r, b2):
    ei = edge_index.astype(jnp.int32)
    src = ei[0].reshape(NW, NCHUNK, 1, CHUNK)
    dst = ei[1].reshape(NW, NCHUNK, CHUNK)
    pad = EPADDED - E
    src2 = jnp.concatenate(
        [ei[0], jnp.zeros((pad,), jnp.int32)]).reshape(NW, NCHUNK2, 1, CHUNK2)
    dst2 = jnp.concatenate(
        [ei[1], jnp.full((pad,), NPAD - 1, jnp.int32)]).reshape(
            NW, NCHUNK2, CHUNK2)
    zeros = jnp.zeros((NPAD, D), jnp.float32)
    ident = jnp.arange(PR, dtype=jnp.int32).reshape(1, PR)

    psum1, pcnt = _sc_sums_hist(x, src, dst, zeros, ident)
    h = _tc_layer(psum1, pcnt, x, W1l, W1r, b1, relu=True)
    (psum2,) = _sc_sums(h, src2, dst2, zeros, ident)
    out = _tc_layer(psum2, pcnt, h, W2l, W2r, b2, relu=False)
    return out


# speedup vs baseline: 1.0003x; 1.0003x over previous
import dataclasses
import functools

import jax
import jax.numpy as jnp
from jax import lax
from jax.experimental import pallas as pl
from jax.experimental.pallas import tpu as pltpu
from jax.experimental.pallas import tpu_sc as plsc

N = 10000
D = 128
E = 320000
NC = 2
NS = 16
NW = NC * NS
EPW = E // NW
CHUNK = 80
NCHUNK = EPW // CHUNK
EPW2 = 10240
EPADDED = EPW2 * NW
CHUNK2 = 128
NCHUNK2 = EPW2 // CHUNK2
NPAD = 10240
RPS = NPAD // NS
PR = NPAD // 128
NBUF = 2

_MESH = plsc.VectorSubcoreMesh(core_axis_name="c", subcore_axis_name="s")


def _make_sc_sums(with_hist, CHUNK, NCHUNK):
    def body(x_hbm, src_hbm, dst_hbm, z_hbm, id_hbm, psum_hbm, pcnt_hbm,
             didx_all, sidx, rows, acc, hist, iid, cacc, isem, gsem, ssem):
        cid = lax.axis_index("c")
        sid = lax.axis_index("s")
        wid = sid * NC + cid
        r0 = sid * RPS
        pltpu.sync_copy(z_hbm.at[pl.ds(r0, RPS)], acc.at[pl.ds(r0, RPS)])
        pltpu.sync_copy(dst_hbm.at[wid], didx_all)
        if with_hist:
            pltpu.sync_copy(z_hbm.at[pl.ds(0, PR)], hist)
            pltpu.sync_copy(id_hbm.at[0], iid)

            @pl.when(sid == 0)
            def _():
                pltpu.sync_copy(z_hbm.at[pl.ds(0, PR)], cacc)
        plsc.subcore_barrier()

        ones16 = jnp.full((16,), 1.0, jnp.float32)

        def step(c, b):
            pltpu.make_async_copy(x_hbm.at[sidx[b].at[0]], rows[b],
                                  gsem[b]).wait()
            pltpu.async_copy(rows[b], acc.at[didx_all.at[c]], ssem[b],
                             add=True)

            @pl.when(c + NBUF < NCHUNK)
            def _():
                pltpu.async_copy(src_hbm.at[wid, c + NBUF], sidx[b], isem[b])

            if with_hist:
                @pl.loop(0, CHUNK, step=16)
                def _(k0):
                    v = didx_all[c, pl.ds(k0, 16)]
                    plsc.addupdate_scatter(
                        hist, [lax.shift_right_logical(v, 7),
                               lax.bitwise_and(v, 127)], ones16)

            b1 = (b + 1) % NBUF

            @pl.when(c + 1 < NCHUNK)
            def _():
                @pl.when(c + 1 >= NBUF)
                def _():
                    pltpu.make_async_copy(rows[b1],
                                          acc.at[didx_all.at[c + 1 - NBUF]],
                                          ssem[b1]).wait()
                pltpu.make_async_copy(src_hbm.at[wid, c + 1], sidx[b1],
                                      isem[b1]).wait()
                pltpu.async_copy(x_hbm.at[sidx[b1].at[0]], rows[b1], gsem[b1])

        for k in range(NBUF):
            pltpu.async_copy(src_hbm.at[wid, k], sidx[k], isem[k])
        pltpu.make_async_copy(src_hbm.at[wid, 0], sidx[0], isem[0]).wait()
        pltpu.async_copy(x_hbm.at[sidx[0].at[0]], rows[0], gsem[0])

        NMAIN = (NCHUNK // NBUF) * NBUF

        @pl.loop(0, NMAIN, step=NBUF)
        def _(t):
            for k in range(NBUF):
                step(t + k, k)

        for c in range(NMAIN, NCHUNK):
            @pl.loop(c, c + 1)
            def _(ct, _b=c % NBUF):
                step(ct, _b)

        for c in range(NCHUNK - NBUF, NCHUNK):
            @pl.loop(c, c + 1)
            def _(ct, _b=c % NBUF):
                pltpu.make_async_copy(rows[_b], acc.at[didx_all.at[ct]],
                                      ssem[_b]).wait()

        if with_hist:
            pltpu.sync_copy(hist, cacc.at[iid], add=True)

        plsc.subcore_barrier()
        pltpu.sync_copy(acc.at[pl.ds(r0, RPS)],
                        psum_hbm.at[cid, pl.ds(r0, RPS)])
        if with_hist:
            @pl.when(sid == 0)
            def _():
                pltpu.sync_copy(cacc, pcnt_hbm.at[cid])

    out_type = [jax.ShapeDtypeStruct((NC, NPAD, D), jnp.float32)]
    if with_hist:
        out_type.append(jax.ShapeDtypeStruct((NC, PR, 128), jnp.float32))
    hp = PR if with_hist else 8
    scratch = (
        [pltpu.VMEM((NCHUNK, CHUNK), jnp.int32)]
        + [pltpu.VMEM((1, CHUNK), jnp.int32)] * NBUF
        + [pltpu.VMEM((CHUNK, D), jnp.float32)] * NBUF
        + [pltpu.VMEM_SHARED((NPAD, D), jnp.float32)]
        + [pltpu.VMEM((hp, 128), jnp.float32),
           pltpu.VMEM((hp,), jnp.int32),
           pltpu.VMEM_SHARED((hp, 128), jnp.float32)]
        + [pltpu.SemaphoreType.DMA] * (3 * NBUF)
    )
    cp = pltpu.CompilerParams()
    if with_hist and (
            "needs_layout_passes" in pltpu.CompilerParams.__dataclass_fields__):
        cp = dataclasses.replace(cp, needs_layout_passes=False)

    def fn(x_hbm, src_hbm, dst_hbm, z_hbm, id_hbm, *rest):
        psum_hbm = rest[0]
        rest = rest[1:]
        if with_hist:
            pcnt_hbm = rest[0]
            rest = rest[1:]
        else:
            pcnt_hbm = None
        didx_all = rest[0]
        sidx = rest[1:1 + NBUF]
        rows = rest[1 + NBUF:1 + 2 * NBUF]
        acc, hist, iid, cacc = rest[1 + 2 * NBUF:5 + 2 * NBUF]
        sems = rest[5 + 2 * NBUF:]
        isem = sems[0:NBUF]
        gsem = sems[NBUF:2 * NBUF]
        ssem = sems[2 * NBUF:3 * NBUF]
        body(x_hbm, src_hbm, dst_hbm, z_hbm, id_hbm, psum_hbm, pcnt_hbm,
             didx_all, sidx, rows, acc, hist, iid, cacc, isem, gsem, ssem)

    return pl.kernel(fn, mesh=_MESH, out_type=out_type,
                     scratch_types=scratch, compiler_params=cp)


_sc_sums_hist = _make_sc_sums(True, CHUNK, NCHUNK)
_sc_sums = _make_sc_sums(False, CHUNK2, NCHUNK2)

BLK = 1024


def _tc_layer_body(relu, p_ref, pc_ref, x_ref, wl_ref, wr_ref, b_ref, o_ref):
    s = p_ref[0] + p_ref[1]
    cw = pc_ref[0] + pc_ref[1]
    rep = jnp.broadcast_to(cw[:, None, :], (BLK // 128, 128, 128))
    rep = rep.reshape(BLK, 128)
    row_mod = lax.broadcasted_iota(jnp.int32, (BLK, 128), 0) % 128
    lane = lax.broadcasted_iota(jnp.int32, (BLK, 128), 1)
    cnt = jnp.sum(jnp.where(row_mod == lane, rep, 0.0), axis=1, keepdims=True)
    mean = s / jnp.maximum(cnt, 1.0)
    acc = lax.dot_general(mean, wl_ref[...], (((1,), (1,)), ((), ())),
                          preferred_element_type=jnp.float32)
    acc += lax.dot_general(x_ref[...], wr_ref[...], (((1,), (1,)), ((), ())),
                           preferred_element_type=jnp.float32)
    acc += b_ref[...]
    o_ref[...] = jnp.maximum(acc, 0.0) if relu else acc


def _tc_layer(psum, pcnt, x, Wl, Wr, b, relu):
    grid = (NPAD // BLK,)
    return pl.pallas_call(
        functools.partial(_tc_layer_body, relu),
        grid=grid,
        in_specs=[
            pl.BlockSpec((NC, BLK, D), lambda i: (0, i, 0)),
            pl.BlockSpec((NC, BLK // 128, 128), lambda i: (0, i, 0)),
            pl.BlockSpec((BLK, D), lambda i: (i, 0)),
            pl.BlockSpec((D, D), lambda i: (0, 0)),
            pl.BlockSpec((D, D), lambda i: (0, 0)),
            pl.BlockSpec((1, D), lambda i: (0, 0)),
        ],
        out_specs=pl.BlockSpec((BLK, D), lambda i: (i, 0)),
        out_shape=jax.ShapeDtypeStruct((N, D), jnp.float32),
    )(psum, pcnt, x, Wl, Wr, b.reshape(1, D))


def kernel(x, edge_index, W1l, W1r, b1, W2l, W2r, b2):
    ei = edge_index.astype(jnp.int32)
    src = ei[0].reshape(NW, NCHUNK, 1, CHUNK)
    dst = ei[1].reshape(NW, NCHUNK, CHUNK)
    pad = EPADDED - E
    src2 = jnp.concatenate(
        [ei[0], jnp.zeros((pad,), jnp.int32)]).reshape(NW, NCHUNK2, 1, CHUNK2)
    dummy_dst = N + (jnp.arange(pad, dtype=jnp.int32) % (NPAD - N))
    dst2 = jnp.concatenate([ei[1], dummy_dst]).reshape(NW, NCHUNK2, CHUNK2)
    zeros = jnp.zeros((NPAD, D), jnp.float32)
    ident = jnp.arange(PR, dtype=jnp.int32).reshape(1, PR)

    psum1, pcnt = _sc_sums_hist(x, src, dst, zeros, ident)
    h = _tc_layer(psum1, pcnt, x, W1l, W1r, b1, relu=True)
    (psum2,) = _sc_sums(h, src2, dst2, zeros, ident)
    out = _tc_layer(psum2, pcnt, h, W2l, W2r, b2, relu=False)
    return out


# speedup vs baseline: 1.7731x; 1.7726x over previous
import dataclasses
import functools

import jax
import jax.numpy as jnp
from jax import lax
from jax.experimental import pallas as pl
from jax.experimental.pallas import tpu as pltpu
from jax.experimental.pallas import tpu_sc as plsc

N = 10000
D = 128
E = 320000
NC = 2
NS = 16
NW = NC * NS
EPW = E // NW
CHUNK = 80
NCHUNK = EPW // CHUNK
NPAD = 10240
RPS = NPAD // NS
PR = NPAD // 128
NBUF = 2

_MESH = plsc.VectorSubcoreMesh(core_axis_name="c", subcore_axis_name="s")


def _make_sc_sums(with_hist):
    def body(x_hbm, src_hbm, dst_hbm, z_hbm, id_hbm, psum_hbm, pcnt_hbm,
             didx_all, sidx, rows, acc, hist, iid, cacc, isem, gsem, ssem):
        cid = lax.axis_index("c")
        sid = lax.axis_index("s")
        wid = sid * NC + cid
        r0 = sid * RPS
        pltpu.sync_copy(z_hbm.at[pl.ds(r0, RPS)], acc.at[pl.ds(r0, RPS)])
        pltpu.sync_copy(dst_hbm.at[wid], didx_all)
        if with_hist:
            pltpu.sync_copy(z_hbm.at[pl.ds(0, PR)], hist)
            pltpu.sync_copy(id_hbm.at[0], iid)

            @pl.when(sid == 0)
            def _():
                pltpu.sync_copy(z_hbm.at[pl.ds(0, PR)], cacc)
        plsc.subcore_barrier()

        ones16 = jnp.full((16,), 1.0, jnp.float32)

        def step(c, b):
            pltpu.make_async_copy(x_hbm.at[sidx[b].at[0]], rows[b],
                                  gsem[b]).wait()
            pltpu.async_copy(rows[b], acc.at[didx_all.at[c]], ssem[b],
                             add=True)

            @pl.when(c + NBUF < NCHUNK)
            def _():
                pltpu.async_copy(src_hbm.at[wid, c + NBUF], sidx[b], isem[b])

            if with_hist:
                @pl.loop(0, CHUNK, step=16)
                def _(k0):
                    v = didx_all[c, pl.ds(k0, 16)]
                    plsc.addupdate_scatter(
                        hist, [lax.shift_right_logical(v, 7),
                               lax.bitwise_and(v, 127)], ones16)

            b1 = (b + 1) % NBUF

            @pl.when(c + 1 < NCHUNK)
            def _():
                @pl.when(c + 1 >= NBUF)
                def _():
                    pltpu.make_async_copy(rows[b1],
                                          acc.at[didx_all.at[c + 1 - NBUF]],
                                          ssem[b1]).wait()
                pltpu.make_async_copy(src_hbm.at[wid, c + 1], sidx[b1],
                                      isem[b1]).wait()
                pltpu.async_copy(x_hbm.at[sidx[b1].at[0]], rows[b1], gsem[b1])

        for k in range(NBUF):
            pltpu.async_copy(src_hbm.at[wid, k], sidx[k], isem[k])
        pltpu.make_async_copy(src_hbm.at[wid, 0], sidx[0], isem[0]).wait()
        pltpu.async_copy(x_hbm.at[sidx[0].at[0]], rows[0], gsem[0])

        NMAIN = (NCHUNK // NBUF) * NBUF

        @pl.loop(0, NMAIN, step=NBUF)
        def _(t):
            for k in range(NBUF):
                step(t + k, k)

        for c in range(NMAIN, NCHUNK):
            @pl.loop(c, c + 1)
            def _(ct, _b=c % NBUF):
                step(ct, _b)

        for c in range(NCHUNK - NBUF, NCHUNK):
            @pl.loop(c, c + 1)
            def _(ct, _b=c % NBUF):
                pltpu.make_async_copy(rows[_b], acc.at[didx_all.at[ct]],
                                      ssem[_b]).wait()

        if with_hist:
            pltpu.sync_copy(hist, cacc.at[iid], add=True)

        plsc.subcore_barrier()
        pltpu.sync_copy(acc.at[pl.ds(r0, RPS)],
                        psum_hbm.at[cid, pl.ds(r0, RPS)])
        if with_hist:
            @pl.when(sid == 0)
            def _():
                pltpu.sync_copy(cacc, pcnt_hbm.at[cid])

    out_type = [jax.ShapeDtypeStruct((NC, NPAD, D), jnp.float32)]
    if with_hist:
        out_type.append(jax.ShapeDtypeStruct((NC, PR, 128), jnp.float32))
    hp = PR if with_hist else 8
    scratch = (
        [pltpu.VMEM((NCHUNK, CHUNK), jnp.int32)]
        + [pltpu.VMEM((1, CHUNK), jnp.int32)] * NBUF
        + [pltpu.VMEM((CHUNK, D), jnp.float32)] * NBUF
        + [pltpu.VMEM_SHARED((NPAD, D), jnp.float32)]
        + [pltpu.VMEM((hp, 128), jnp.float32),
           pltpu.VMEM((hp,), jnp.int32),
           pltpu.VMEM_SHARED((hp, 128), jnp.float32)]
        + [pltpu.SemaphoreType.DMA] * (3 * NBUF)
    )
    cp = pltpu.CompilerParams()
    if with_hist and (
            "needs_layout_passes" in pltpu.CompilerParams.__dataclass_fields__):
        cp = dataclasses.replace(cp, needs_layout_passes=False)

    def fn(x_hbm, src_hbm, dst_hbm, z_hbm, id_hbm, *rest):
        psum_hbm = rest[0]
        rest = rest[1:]
        if with_hist:
            pcnt_hbm = rest[0]
            rest = rest[1:]
        else:
            pcnt_hbm = None
        didx_all = rest[0]
        sidx = rest[1:1 + NBUF]
        rows = rest[1 + NBUF:1 + 2 * NBUF]
        acc, hist, iid, cacc = rest[1 + 2 * NBUF:5 + 2 * NBUF]
        sems = rest[5 + 2 * NBUF:]
        isem = sems[0:NBUF]
        gsem = sems[NBUF:2 * NBUF]
        ssem = sems[2 * NBUF:3 * NBUF]
        body(x_hbm, src_hbm, dst_hbm, z_hbm, id_hbm, psum_hbm, pcnt_hbm,
             didx_all, sidx, rows, acc, hist, iid, cacc, isem, gsem, ssem)

    return pl.kernel(fn, mesh=_MESH, out_type=out_type,
                     scratch_types=scratch, compiler_params=cp)


_sc_sums_hist = _make_sc_sums(True)
_sc_sums = _make_sc_sums(False)

BLK = 1024


def _tc_layer_body(relu, p_ref, pc_ref, x_ref, wl_ref, wr_ref, b_ref, o_ref):
    s = p_ref[0] + p_ref[1]
    cw = pc_ref[0] + pc_ref[1]
    rep = jnp.broadcast_to(cw[:, None, :], (BLK // 128, 128, 128))
    rep = rep.reshape(BLK, 128)
    row_mod = lax.broadcasted_iota(jnp.int32, (BLK, 128), 0) % 128
    lane = lax.broadcasted_iota(jnp.int32, (BLK, 128), 1)
    cnt = jnp.sum(jnp.where(row_mod == lane, rep, 0.0), axis=1, keepdims=True)
    mean = s / jnp.maximum(cnt, 1.0)
    acc = lax.dot_general(mean, wl_ref[...], (((1,), (1,)), ((), ())),
                          preferred_element_type=jnp.float32)
    acc += lax.dot_general(x_ref[...], wr_ref[...], (((1,), (1,)), ((), ())),
                           preferred_element_type=jnp.float32)
    acc += b_ref[...]
    o_ref[...] = jnp.maximum(acc, 0.0) if relu else acc


def _tc_layer(psum, pcnt, x, Wl, Wr, b, relu):
    grid = (NPAD // BLK,)
    return pl.pallas_call(
        functools.partial(_tc_layer_body, relu),
        grid=grid,
        in_specs=[
            pl.BlockSpec((NC, BLK, D), lambda i: (0, i, 0)),
            pl.BlockSpec((NC, BLK // 128, 128), lambda i: (0, i, 0)),
            pl.BlockSpec((BLK, D), lambda i: (i, 0)),
            pl.BlockSpec((D, D), lambda i: (0, 0)),
            pl.BlockSpec((D, D), lambda i: (0, 0)),
            pl.BlockSpec((1, D), lambda i: (0, 0)),
        ],
        out_specs=pl.BlockSpec((BLK, D), lambda i: (i, 0)),
        out_shape=jax.ShapeDtypeStruct((N, D), jnp.float32),
    )(psum, pcnt, x, Wl, Wr, b.reshape(1, D))


def kernel(x, edge_index, W1l, W1r, b1, W2l, W2r, b2):
    ei = edge_index.astype(jnp.int32)
    src = ei[0].reshape(NW, NCHUNK, 1, CHUNK)
    dst = ei[1].reshape(NW, NCHUNK, CHUNK)
    zeros = jnp.zeros((NPAD, D), jnp.float32)
    ident = jnp.arange(PR, dtype=jnp.int32).reshape(1, PR)

    psum1, pcnt = _sc_sums_hist(x, src, dst, zeros, ident)
    h = _tc_layer(psum1, pcnt, x, W1l, W1r, b1, relu=True)
    (psum2,) = _sc_sums(h, src, dst, zeros, ident)
    out = _tc_layer(psum2, pcnt, h, W2l, W2r, b2, relu=False)
    return out
